# Initial kernel scaffold; baseline (speedup 1.0000x reference)
import jax
import jax.numpy as jnp
from jax.experimental import pallas as pl


def kernel(x, edge_index, initial_coords, W_node, b_node, W_coord, b_coord, W_gcn, b_gcn, W_force, b_force):
    raise NotImplementedError("write your pallas kernel here")


# speedup vs baseline: 16.9912x; 16.9912x over previous
import functools

import jax
import jax.numpy as jnp
from jax import lax
from jax.experimental import pallas as pl
from jax.experimental.pallas import tpu as pltpu
from jax.experimental.pallas import tpu_sc as plsc

N = 10000
E = 320000
D_IN = 128
H = 128
HC = 32

NC = 2
NS = 16
NW = NC * NS
EDGES_PER_W = E // NW
CHUNK = 80
STEPS = EDGES_PER_W // CHUNK
NACC = 10240
RPT = NACC // NS
RCOPY = RPT // CHUNK

_mesh = plsc.VectorSubcoreMesh(
    core_axis_name="c", subcore_axis_name="s", num_cores=NC, num_subcores=NS
)


@functools.partial(
    pl.kernel,
    out_type=jax.ShapeDtypeStruct((NC * NACC,), jnp.float32),
    mesh=_mesh,
    scratch_types=[
        pltpu.VMEM((CHUNK,), jnp.int32),
        pltpu.VMEM((CHUNK,), jnp.float32),
        pltpu.VMEM((RPT,), jnp.float32),
        pltpu.VMEM_SHARED((NACC,), jnp.float32),
    ],
)
def _sc_deg(dst_hbm, out_hbm, idx_v, ones_v, buf_v, deg_sh):
    c = lax.axis_index("c")
    s = lax.axis_index("s")
    w = s * NC + c

    for j in range(CHUNK // 16):
        ones_v[pl.ds(j * 16, 16)] = jnp.ones((16,), jnp.float32)

    def zbody(r, _):
        buf_v[pl.ds(r * 16, 16)] = jnp.zeros((16,), jnp.float32)
        return 0

    lax.fori_loop(0, RPT // 16, zbody, 0)
    pltpu.sync_copy(buf_v, deg_sh.at[pl.ds(s * RPT, RPT)])
    plsc.subcore_barrier()

    def body(i, _):
        base = w * EDGES_PER_W + i * CHUNK
        pltpu.sync_copy(dst_hbm.at[pl.ds(base, CHUNK)], idx_v)
        pltpu.sync_copy(ones_v, deg_sh.at[idx_v], add=True)
        return 0

    lax.fori_loop(0, STEPS, body, 0)
    plsc.subcore_barrier()

    pltpu.sync_copy(deg_sh.at[pl.ds(s * RPT, RPT)], buf_v)
    pltpu.sync_copy(buf_v, out_hbm.at[pl.ds(c * NACC + s * RPT, RPT)])


@functools.partial(
    pl.kernel,
    out_type=jax.ShapeDtypeStruct((NC * NACC, H), jnp.float32),
    mesh=_mesh,
    scratch_types=[
        pltpu.VMEM((CHUNK,), jnp.int32),
        pltpu.VMEM((CHUNK,), jnp.int32),
        pltpu.VMEM((CHUNK, H), jnp.float32),
        pltpu.SemaphoreType.DMA,
        pltpu.VMEM_SHARED((NACC, H), jnp.float32),
    ],
)
def _sc_scatter(g_hbm, src_hbm, dst_hbm, out_hbm,
                src_v, dst_v, rows_v, sem, acc_sh):
    c = lax.axis_index("c")
    s = lax.axis_index("s")
    w = s * NC + c

    def zbody(r, _):
        for j in range(H // 16):
            rows_v[r, pl.ds(j * 16, 16)] = jnp.zeros((16,), jnp.float32)
        return 0

    lax.fori_loop(0, CHUNK, zbody, 0)
    for k in range(RCOPY):
        pltpu.sync_copy(rows_v, acc_sh.at[pl.ds(s * RPT + k * CHUNK, CHUNK)])
    plsc.subcore_barrier()

    def body(i, _):
        base = w * EDGES_PER_W + i * CHUNK
        pltpu.sync_copy(src_hbm.at[pl.ds(base, CHUNK)], src_v)
        pltpu.async_copy(g_hbm.at[src_v], rows_v, sem).wait()
        pltpu.sync_copy(dst_hbm.at[pl.ds(base, CHUNK)], dst_v)
        pltpu.sync_copy(rows_v, acc_sh.at[dst_v], add=True)
        return 0

    lax.fori_loop(0, STEPS, body, 0)
    plsc.subcore_barrier()

    for k in range(RCOPY):
        pltpu.sync_copy(acc_sh.at[pl.ds(s * RPT + k * CHUNK, CHUNK)], rows_v)
        pltpu.sync_copy(
            rows_v,
            out_hbm.at[pl.ds(c * NACC + s * RPT + k * CHUNK, CHUNK)],
        )


_BN = 1000


def _tc_g_body(x_ref, co_ref, d0_ref, d1_ref, wn_ref, bn_ref, wc_ref, bc_ref,
               wg1_ref, wg2_ref, g_ref):
    hn = jnp.dot(x_ref[...], wn_ref[...], preferred_element_type=jnp.float32)
    hn = hn + bn_ref[...]
    hc = jnp.dot(co_ref[...], wc_ref[...], preferred_element_type=jnp.float32)
    hc = hc + bc_ref[...]
    hw = (jnp.dot(hn, wg1_ref[...], preferred_element_type=jnp.float32)
          + jnp.dot(hc, wg2_ref[...], preferred_element_type=jnp.float32))
    dinv = lax.rsqrt(d0_ref[...] + d1_ref[...] + 1.0)
    g_ref[...] = dinv * hw


def _tc_final_body(a0_ref, a1_ref, g_ref, d0_ref, d1_ref, bg_ref, wf_ref,
                   bf_ref, out_ref):
    ssum = a0_ref[...] + a1_ref[...] + g_ref[...]
    dinv = lax.rsqrt(d0_ref[...] + d1_ref[...] + 1.0)
    h = jnp.maximum(dinv * ssum + bg_ref[...], 0.0)
    out_ref[...] = (jnp.dot(h, wf_ref[...], preferred_element_type=jnp.float32)
                    + bf_ref[...])


def _row_block(bn, cols):
    return pl.BlockSpec((bn, cols), lambda i: (i, 0))


def _whole(shape):
    return pl.BlockSpec(shape, lambda i: tuple(0 for _ in shape))


def kernel(x, edge_index, initial_coords, W_node, b_node, W_coord, b_coord,
           W_gcn, b_gcn, W_force, b_force):
    src = edge_index[0]
    dst = edge_index[1]

    deg_p = _sc_deg(dst)
    d0 = deg_p[:N].reshape(N, 1)
    d1 = deg_p[NACC:NACC + N].reshape(N, 1)

    g = pl.pallas_call(
        _tc_g_body,
        grid=(N // _BN,),
        in_specs=[
            _row_block(_BN, D_IN),
            _row_block(_BN, 2),
            _row_block(_BN, 1),
            _row_block(_BN, 1),
            _whole((D_IN, H)),
            _whole((1, H)),
            _whole((2, HC)),
            _whole((1, HC)),
            _whole((H, H)),
            _whole((HC, H)),
        ],
        out_specs=_row_block(_BN, H),
        out_shape=jax.ShapeDtypeStruct((N, H), jnp.float32),
    )(x, initial_coords, d0, d1, W_node, b_node.reshape(1, H),
      W_coord, b_coord.reshape(1, HC), W_gcn[:H], W_gcn[H:])

    acc_p = _sc_scatter(g, src, dst)
    a0 = acc_p[:N]
    a1 = acc_p[NACC:NACC + N]

    out = pl.pallas_call(
        _tc_final_body,
        grid=(N // _BN,),
        in_specs=[
            _row_block(_BN, H),
            _row_block(_BN, H),
            _row_block(_BN, H),
            _row_block(_BN, 1),
            _row_block(_BN, 1),
            _whole((1, H)),
            _whole((H, 2)),
            _whole((1, 2)),
        ],
        out_specs=_row_block(_BN, 2),
        out_shape=jax.ShapeDtypeStruct((N, 2), jnp.float32),
    )(a0, a1, g, d0, d1, b_gcn.reshape(1, H), W_force, b_force.reshape(1, 2))

    return out


# speedup vs baseline: 34.9256x; 2.0555x over previous
import functools

import jax
import jax.numpy as jnp
from jax import lax
from jax.experimental import pallas as pl
from jax.experimental.pallas import tpu as pltpu
from jax.experimental.pallas import tpu_sc as plsc

N = 10000
E = 320000
D_IN = 128
H = 128
HC = 32

NC = 2
NS = 16
NW = NC * NS
EDGES_PER_W = E // NW
CHUNK = 80
STEPS = EDGES_PER_W // CHUNK
NACC = 10240
RPT = NACC // NS
RCOPY = RPT // CHUNK

_mesh = plsc.VectorSubcoreMesh(
    core_axis_name="c", subcore_axis_name="s", num_cores=NC, num_subcores=NS
)


KBUF = 5
ROUNDS = STEPS // KBUF
KS = 3
ROUNDS_S = 40


@functools.partial(
    pl.kernel,
    out_type=jax.ShapeDtypeStruct((NC * NACC,), jnp.float32),
    mesh=_mesh,
    scratch_types=(
        [pltpu.VMEM((CHUNK,), jnp.float32), pltpu.VMEM((RPT,), jnp.float32)]
        + [pltpu.VMEM((CHUNK,), jnp.int32)] * KBUF
        + [pltpu.SemaphoreType.DMA] * (2 * KBUF)
        + [pltpu.VMEM_SHARED((NACC,), jnp.float32)]
    ),
)
def _sc_deg(dst_hbm, out_hbm, ones_v, buf_v, *rest):
    dstb = rest[:KBUF]
    lsem = rest[KBUF:2 * KBUF]
    ssem = rest[2 * KBUF:3 * KBUF]
    deg_sh = rest[3 * KBUF]

    c = lax.axis_index("c")
    s = lax.axis_index("s")
    w = s * NC + c
    base0 = w * EDGES_PER_W

    for j in range(CHUNK // 16):
        ones_v[pl.ds(j * 16, 16)] = jnp.ones((16,), jnp.float32)

    def zbody(r, _):
        buf_v[pl.ds(r * 16, 16)] = jnp.zeros((16,), jnp.float32)
        return 0

    lax.fori_loop(0, RPT // 16, zbody, 0)
    pltpu.sync_copy(buf_v, deg_sh.at[pl.ds(s * RPT, RPT)])
    plsc.subcore_barrier()

    for b in range(KBUF):
        pltpu.async_copy(dst_hbm.at[pl.ds(base0 + b * CHUNK, CHUNK)],
                         dstb[b], lsem[b])

    def body(j, _):
        i0 = j * KBUF
        for b in range(KBUF):
            pltpu.make_async_copy(dst_hbm.at[pl.ds(base0, CHUNK)], dstb[b],
                                  lsem[b]).wait()
            pltpu.async_copy(ones_v, deg_sh.at[dstb[b]], ssem[b], add=True)
        for b in range(KBUF):
            i = i0 + b

            @pl.when(j < ROUNDS - 1)
            def _():
                pltpu.make_async_copy(ones_v, deg_sh.at[dstb[b]],
                                      ssem[b]).wait()
                pltpu.async_copy(
                    dst_hbm.at[pl.ds(base0 + (i + KBUF) * CHUNK, CHUNK)],
                    dstb[b], lsem[b])
        return 0

    lax.fori_loop(0, ROUNDS, body, 0)
    for b in range(KBUF):
        pltpu.make_async_copy(ones_v, deg_sh.at[dstb[b]], ssem[b]).wait()
    plsc.subcore_barrier()

    pltpu.sync_copy(deg_sh.at[pl.ds(s * RPT, RPT)], buf_v)
    pltpu.sync_copy(buf_v, out_hbm.at[pl.ds(c * NACC + s * RPT, RPT)])


@functools.partial(
    pl.kernel,
    out_type=jax.ShapeDtypeStruct((NC * NACC, H), jnp.float32),
    mesh=_mesh,
    scratch_types=(
        [pltpu.VMEM((STEPS, CHUNK), jnp.int32)]
        + [pltpu.VMEM((CHUNK,), jnp.int32)] * KS
        + [pltpu.VMEM((CHUNK, H), jnp.float32)] * KS
        + [pltpu.SemaphoreType.DMA] * (3 * KS)
        + [pltpu.VMEM_SHARED((NACC, H), jnp.float32)]
    ),
)
def _sc_scatter(g_hbm, src_hbm, dst_hbm, out_hbm, src_all, *rest):
    dstb = rest[:KS]
    rows = rest[KS:2 * KS]
    gsem = rest[2 * KS:3 * KS]
    lsem = rest[3 * KS:4 * KS]
    ssem = rest[4 * KS:5 * KS]
    acc_sh = rest[5 * KS]

    c = lax.axis_index("c")
    s = lax.axis_index("s")
    w = s * NC + c
    base0 = w * EDGES_PER_W

    pltpu.sync_copy(src_hbm.at[w], src_all)

    def zbody(r, _):
        for j in range(H // 16):
            rows[0][r, pl.ds(j * 16, 16)] = jnp.zeros((16,), jnp.float32)
        return 0

    lax.fori_loop(0, CHUNK, zbody, 0)
    for k in range(RCOPY):
        pltpu.sync_copy(rows[0], acc_sh.at[pl.ds(s * RPT + k * CHUNK, CHUNK)])
    plsc.subcore_barrier()

    for b in range(KS):
        pltpu.async_copy(g_hbm.at[src_all.at[b]], rows[b], gsem[b])
        pltpu.async_copy(dst_hbm.at[pl.ds(base0 + b * CHUNK, CHUNK)],
                         dstb[b], lsem[b])

    def body(j, _):
        i0 = j * KS
        for b in range(KS):
            i = i0 + b
            pltpu.make_async_copy(g_hbm.at[src_all.at[i]], rows[b],
                                  gsem[b]).wait()
            pltpu.make_async_copy(dst_hbm.at[pl.ds(base0, CHUNK)], dstb[b],
                                  lsem[b]).wait()
            pltpu.async_copy(rows[b], acc_sh.at[dstb[b]], ssem[b], add=True)
        for b in range(KS):
            i = i0 + b

            @pl.when(j < ROUNDS_S - 1)
            def _():
                pltpu.make_async_copy(rows[b], acc_sh.at[dstb[b]],
                                      ssem[b]).wait()
                pltpu.async_copy(g_hbm.at[src_all.at[i + KS]], rows[b],
                                 gsem[b])
                pltpu.async_copy(
                    dst_hbm.at[pl.ds(base0 + (i + KS) * CHUNK, CHUNK)],
                    dstb[b], lsem[b])
        return 0

    lax.fori_loop(0, ROUNDS_S, body, 0)

    for i in range(KS * ROUNDS_S, STEPS):
        b = (i - KS * ROUNDS_S) % KS
        pltpu.make_async_copy(rows[b], acc_sh.at[dstb[b]], ssem[b]).wait()
        pltpu.async_copy(g_hbm.at[src_all.at[i]], rows[b], gsem[b])
        pltpu.async_copy(dst_hbm.at[pl.ds(base0 + i * CHUNK, CHUNK)],
                         dstb[b], lsem[b])
        pltpu.make_async_copy(g_hbm.at[src_all.at[i]], rows[b],
                              gsem[b]).wait()
        pltpu.make_async_copy(dst_hbm.at[pl.ds(base0, CHUNK)], dstb[b],
                              lsem[b]).wait()
        pltpu.async_copy(rows[b], acc_sh.at[dstb[b]], ssem[b], add=True)

    for b in range(KS):
        pltpu.make_async_copy(rows[b], acc_sh.at[dstb[b]], ssem[b]).wait()
    plsc.subcore_barrier()

    for k in range(RCOPY):
        pltpu.sync_copy(acc_sh.at[pl.ds(s * RPT + k * CHUNK, CHUNK)], rows[0])
        pltpu.sync_copy(
            rows[0],
            out_hbm.at[pl.ds(c * NACC + s * RPT + k * CHUNK, CHUNK)],
        )


_BN = 1000


def _tc_g_body(x_ref, co_ref, d0_ref, d1_ref, wn_ref, bn_ref, wc_ref, bc_ref,
               wg1_ref, wg2_ref, g_ref):
    hn = jnp.dot(x_ref[...], wn_ref[...], preferred_element_type=jnp.float32)
    hn = hn + bn_ref[...]
    hc = jnp.dot(co_ref[...], wc_ref[...], preferred_element_type=jnp.float32)
    hc = hc + bc_ref[...]
    hw = (jnp.dot(hn, wg1_ref[...], preferred_element_type=jnp.float32)
          + jnp.dot(hc, wg2_ref[...], preferred_element_type=jnp.float32))
    dinv = lax.rsqrt(d0_ref[...] + d1_ref[...] + 1.0)
    g_ref[...] = dinv * hw


def _tc_final_body(a0_ref, a1_ref, g_ref, d0_ref, d1_ref, bg_ref, wf_ref,
                   bf_ref, out_ref):
    ssum = a0_ref[...] + a1_ref[...] + g_ref[...]
    dinv = lax.rsqrt(d0_ref[...] + d1_ref[...] + 1.0)
    h = jnp.maximum(dinv * ssum + bg_ref[...], 0.0)
    out_ref[...] = (jnp.dot(h, wf_ref[...], preferred_element_type=jnp.float32)
                    + bf_ref[...])


def _row_block(bn, cols):
    return pl.BlockSpec((bn, cols), lambda i: (i, 0))


def _whole(shape):
    return pl.BlockSpec(shape, lambda i: tuple(0 for _ in shape))


def kernel(x, edge_index, initial_coords, W_node, b_node, W_coord, b_coord,
           W_gcn, b_gcn, W_force, b_force):
    src = edge_index[0].reshape(NW, STEPS, CHUNK)
    dst = edge_index[1]

    deg_p = _sc_deg(dst)
    d0 = deg_p[:N].reshape(N, 1)
    d1 = deg_p[NACC:NACC + N].reshape(N, 1)

    g = pl.pallas_call(
        _tc_g_body,
        grid=(N // _BN,),
        in_specs=[
            _row_block(_BN, D_IN),
            _row_block(_BN, 2),
            _row_block(_BN, 1),
            _row_block(_BN, 1),
            _whole((D_IN, H)),
            _whole((1, H)),
            _whole((2, HC)),
            _whole((1, HC)),
            _whole((H, H)),
            _whole((HC, H)),
        ],
        out_specs=_row_block(_BN, H),
        out_shape=jax.ShapeDtypeStruct((N, H), jnp.float32),
    )(x, initial_coords, d0, d1, W_node, b_node.reshape(1, H),
      W_coord, b_coord.reshape(1, HC), W_gcn[:H], W_gcn[H:])

    acc_p = _sc_scatter(g, src, dst)
    a0 = acc_p[:N]
    a1 = acc_p[NACC:NACC + N]

    out = pl.pallas_call(
        _tc_final_body,
        grid=(N // _BN,),
        in_specs=[
            _row_block(_BN, H),
            _row_block(_BN, H),
            _row_block(_BN, H),
            _row_block(_BN, 1),
            _row_block(_BN, 1),
            _whole((1, H)),
            _whole((H, 2)),
            _whole((1, 2)),
        ],
        out_specs=_row_block(_BN, 2),
        out_shape=jax.ShapeDtypeStruct((N, 2), jnp.float32),
    )(a0, a1, g, d0, d1, b_gcn.reshape(1, H), W_force, b_force.reshape(1, 2))

    return out


# speedup vs baseline: 36.5638x; 1.0469x over previous
import functools

import jax
import jax.numpy as jnp
from jax import lax
from jax.experimental import pallas as pl
from jax.experimental.pallas import tpu as pltpu
from jax.experimental.pallas import tpu_sc as plsc

N = 10000
E = 320000
D_IN = 128
H = 128
HC = 32

NC = 2
NS = 16
NW = NC * NS
EDGES_PER_W = E // NW
CHUNK = 80
STEPS = EDGES_PER_W // CHUNK
NACC = 10240
RPT = NACC // NS
RCOPY = RPT // CHUNK
LASTC = (N - (NS - 1) * RPT) // CHUNK

KBUF = 5
ROUNDS = STEPS // KBUF
KS = 3
ROUNDS_S = 40

_mesh = plsc.VectorSubcoreMesh(
    core_axis_name="c", subcore_axis_name="s", num_cores=NC, num_subcores=NS
)


@functools.partial(
    pl.kernel,
    out_type=jax.ShapeDtypeStruct((NC * N,), jnp.float32),
    mesh=_mesh,
    scratch_types=(
        [pltpu.VMEM((CHUNK,), jnp.float32), pltpu.VMEM((RPT,), jnp.float32)]
        + [pltpu.VMEM((CHUNK,), jnp.int32)] * KBUF
        + [pltpu.SemaphoreType.DMA] * (2 * KBUF)
        + [pltpu.VMEM_SHARED((NACC,), jnp.float32)]
    ),
)
def _sc_deg(dst_hbm, out_hbm, ones_v, buf_v, *rest):
    dstb = rest[:KBUF]
    lsem = rest[KBUF:2 * KBUF]
    ssem = rest[2 * KBUF:3 * KBUF]
    deg_sh = rest[3 * KBUF]

    c = lax.axis_index("c")
    s = lax.axis_index("s")
    w = s * NC + c
    base0 = w * EDGES_PER_W

    for j in range(CHUNK // 16):
        ones_v[pl.ds(j * 16, 16)] = jnp.ones((16,), jnp.float32)

    def zbody(r, _):
        buf_v[pl.ds(r * 16, 16)] = jnp.zeros((16,), jnp.float32)
        return 0

    lax.fori_loop(0, RPT // 16, zbody, 0)
    pltpu.sync_copy(buf_v, deg_sh.at[pl.ds(s * RPT, RPT)])
    plsc.subcore_barrier()

    for b in range(KBUF):
        pltpu.async_copy(dst_hbm.at[pl.ds(base0 + b * CHUNK, CHUNK)],
                         dstb[b], lsem[b])

    def body(j, _):
        i0 = j * KBUF
        for b in range(KBUF):
            pltpu.make_async_copy(dst_hbm.at[pl.ds(base0, CHUNK)], dstb[b],
                                  lsem[b]).wait()
            pltpu.async_copy(ones_v, deg_sh.at[dstb[b]], ssem[b], add=True)
        for b in range(KBUF):
            i = i0 + b

            @pl.when(j < ROUNDS - 1)
            def _():
                pltpu.make_async_copy(ones_v, deg_sh.at[dstb[b]],
                                      ssem[b]).wait()
                pltpu.async_copy(
                    dst_hbm.at[pl.ds(base0 + (i + KBUF) * CHUNK, CHUNK)],
                    dstb[b], lsem[b])
        return 0

    lax.fori_loop(0, ROUNDS, body, 0)
    for b in range(KBUF):
        pltpu.make_async_copy(ones_v, deg_sh.at[dstb[b]], ssem[b]).wait()
    plsc.subcore_barrier()

    pltpu.sync_copy(deg_sh.at[pl.ds(s * RPT, RPT)], buf_v)

    @pl.when(s < NS - 1)
    def _():
        pltpu.sync_copy(buf_v, out_hbm.at[pl.ds(c * N + s * RPT, RPT)])

    @pl.when(s == NS - 1)
    def _():
        pltpu.sync_copy(buf_v.at[pl.ds(0, LASTC * CHUNK)],
                        out_hbm.at[pl.ds(c * N + (NS - 1) * RPT,
                                         LASTC * CHUNK)])


@functools.partial(
    pl.kernel,
    out_type=jax.ShapeDtypeStruct((NC * N, H), jnp.float32),
    mesh=_mesh,
    scratch_types=(
        [pltpu.VMEM((STEPS, CHUNK), jnp.int32)]
        + [pltpu.VMEM((CHUNK,), jnp.int32)] * KS
        + [pltpu.VMEM((CHUNK, H), jnp.float32)] * KS
        + [pltpu.SemaphoreType.DMA] * (3 * KS)
        + [pltpu.VMEM_SHARED((NACC, H), jnp.float32)]
    ),
)
def _sc_scatter(g_hbm, src_hbm, dst_hbm, out_hbm, src_all, *rest):
    dstb = rest[:KS]
    rows = rest[KS:2 * KS]
    gsem = rest[2 * KS:3 * KS]
    lsem = rest[3 * KS:4 * KS]
    ssem = rest[4 * KS:5 * KS]
    acc_sh = rest[5 * KS]

    c = lax.axis_index("c")
    s = lax.axis_index("s")
    w = s * NC + c
    base0 = w * EDGES_PER_W

    pltpu.sync_copy(src_hbm.at[w], src_all)

    def zbody(r, _):
        for j in range(H // 16):
            rows[0][r, pl.ds(j * 16, 16)] = jnp.zeros((16,), jnp.float32)
        return 0

    lax.fori_loop(0, CHUNK, zbody, 0)
    for k in range(RCOPY):
        pltpu.sync_copy(rows[0], acc_sh.at[pl.ds(s * RPT + k * CHUNK, CHUNK)])
    plsc.subcore_barrier()

    for b in range(KS):
        pltpu.async_copy(g_hbm.at[src_all.at[b]], rows[b], gsem[b])
        pltpu.async_copy(dst_hbm.at[pl.ds(base0 + b * CHUNK, CHUNK)],
                         dstb[b], lsem[b])

    def body(j, _):
        i0 = j * KS
        for b in range(KS):
            i = i0 + b
            pltpu.make_async_copy(g_hbm.at[src_all.at[i]], rows[b],
                                  gsem[b]).wait()
            pltpu.make_async_copy(dst_hbm.at[pl.ds(base0, CHUNK)], dstb[b],
                                  lsem[b]).wait()
            pltpu.async_copy(rows[b], acc_sh.at[dstb[b]], ssem[b], add=True)
        for b in range(KS):
            i = i0 + b

            @pl.when(j < ROUNDS_S - 1)
            def _():
                pltpu.make_async_copy(rows[b], acc_sh.at[dstb[b]],
                                      ssem[b]).wait()
                pltpu.async_copy(g_hbm.at[src_all.at[i + KS]], rows[b],
                                 gsem[b])
                pltpu.async_copy(
                    dst_hbm.at[pl.ds(base0 + (i + KS) * CHUNK, CHUNK)],
                    dstb[b], lsem[b])
        return 0

    lax.fori_loop(0, ROUNDS_S, body, 0)

    for i in range(KS * ROUNDS_S, STEPS):
        b = (i - KS * ROUNDS_S) % KS
        pltpu.make_async_copy(rows[b], acc_sh.at[dstb[b]], ssem[b]).wait()
        pltpu.async_copy(g_hbm.at[src_all.at[i]], rows[b], gsem[b])
        pltpu.async_copy(dst_hbm.at[pl.ds(base0 + i * CHUNK, CHUNK)],
                         dstb[b], lsem[b])
        pltpu.make_async_copy(g_hbm.at[src_all.at[i]], rows[b],
                              gsem[b]).wait()
        pltpu.make_async_copy(dst_hbm.at[pl.ds(base0, CHUNK)], dstb[b],
                              lsem[b]).wait()
        pltpu.async_copy(rows[b], acc_sh.at[dstb[b]], ssem[b], add=True)

    for b in range(KS):
        pltpu.make_async_copy(rows[b], acc_sh.at[dstb[b]], ssem[b]).wait()
    plsc.subcore_barrier()

    for k in range(RCOPY):
        if k < LASTC:
            pltpu.sync_copy(acc_sh.at[pl.ds(s * RPT + k * CHUNK, CHUNK)],
                            rows[0])
            pltpu.sync_copy(
                rows[0],
                out_hbm.at[pl.ds(c * N + s * RPT + k * CHUNK, CHUNK)],
            )
        else:
            @pl.when(s < NS - 1)
            def _():
                pltpu.sync_copy(acc_sh.at[pl.ds(s * RPT + k * CHUNK, CHUNK)],
                                rows[0])
                pltpu.sync_copy(
                    rows[0],
                    out_hbm.at[pl.ds(c * N + s * RPT + k * CHUNK, CHUNK)],
                )


_BN = 1000
_NB = N // _BN


def _tc_hw_body(x_ref, co_ref, wn_ref, bn_ref, wc_ref, bc_ref,
                wg1_ref, wg2_ref, hw_ref):
    hn = jnp.dot(x_ref[...], wn_ref[...], preferred_element_type=jnp.float32)
    hn = hn + bn_ref[...]
    hc = jnp.dot(co_ref[...], wc_ref[...], preferred_element_type=jnp.float32)
    hc = hc + bc_ref[...]
    hw_ref[...] = (jnp.dot(hn, wg1_ref[...], preferred_element_type=jnp.float32)
                   + jnp.dot(hc, wg2_ref[...], preferred_element_type=jnp.float32))


def _tc_scale_body(hw_ref, d0_ref, d1_ref, g_ref):
    dinv = lax.rsqrt(d0_ref[...] + d1_ref[...] + 1.0)
    g_ref[...] = dinv * hw_ref[...]


def _tc_final_body(a0_ref, a1_ref, g_ref, d0_ref, d1_ref, bg_ref, wf_ref,
                   bf_ref, out_ref):
    ssum = a0_ref[...] + a1_ref[...] + g_ref[...]
    dinv = lax.rsqrt(d0_ref[...] + d1_ref[...] + 1.0)
    h = jnp.maximum(dinv * ssum + bg_ref[...], 0.0)
    out_ref[...] = (jnp.dot(h, wf_ref[...], preferred_element_type=jnp.float32)
                    + bf_ref[...])


def _row_block(cols, off=0):
    return pl.BlockSpec((_BN, cols), lambda i, o=off: (i + o, 0))


def _whole(shape):
    return pl.BlockSpec(shape, lambda i: tuple(0 for _ in shape))


def kernel(x, edge_index, initial_coords, W_node, b_node, W_coord, b_coord,
           W_gcn, b_gcn, W_force, b_force):
    src = edge_index[0].reshape(NW, STEPS, CHUNK)
    dst = edge_index[1]

    deg_p = _sc_deg(dst).reshape(NC * N, 1)

    hw = pl.pallas_call(
        _tc_hw_body,
        grid=(_NB,),
        in_specs=[
            _row_block(D_IN),
            _row_block(2),
            _whole((D_IN, H)),
            _whole((1, H)),
            _whole((2, HC)),
            _whole((1, HC)),
            _whole((H, H)),
            _whole((HC, H)),
        ],
        out_specs=_row_block(H),
        out_shape=jax.ShapeDtypeStruct((N, H), jnp.float32),
    )(x, initial_coords, W_node, b_node.reshape(1, H),
      W_coord, b_coord.reshape(1, HC), W_gcn[:H], W_gcn[H:])

    g = pl.pallas_call(
        _tc_scale_body,
        grid=(_NB,),
        in_specs=[
            _row_block(H),
            _row_block(1),
            _row_block(1, off=_NB),
        ],
        out_specs=_row_block(H),
        out_shape=jax.ShapeDtypeStruct((N, H), jnp.float32),
    )(hw, deg_p, deg_p)

    acc_p = _sc_scatter(g, src, dst)

    out = pl.pallas_call(
        _tc_final_body,
        grid=(_NB,),
        in_specs=[
            _row_block(H),
            _row_block(H, off=_NB),
            _row_block(H),
            _row_block(1),
            _row_block(1, off=_NB),
            _whole((1, H)),
            _whole((H, 2)),
            _whole((1, 2)),
        ],
        out_specs=_row_block(2),
        out_shape=jax.ShapeDtypeStruct((N, 2), jnp.float32),
    )(acc_p, acc_p, g, deg_p, deg_p, b_gcn.reshape(1, H), W_force,
      b_force.reshape(1, 2))

    return out


# speedup vs baseline: 40.3109x; 1.1025x over previous
import functools

import jax
import jax.numpy as jnp
from jax import lax
from jax.experimental import pallas as pl
from jax.experimental.pallas import tpu as pltpu
from jax.experimental.pallas import tpu_sc as plsc

N = 10000
E = 320000
D_IN = 128
H = 128
HC = 32

NC = 2
NS = 16
NW = NC * NS
EDGES_PER_W = E // NW
CHUNK = 80
STEPS = EDGES_PER_W // CHUNK
NACC = 10240
RPT = NACC // NS
RCOPY = RPT // CHUNK
LASTC = (N - (NS - 1) * RPT) // CHUNK

KBUF = 5
ROUNDS = STEPS // KBUF
KS = 3
ROUNDS_S = 40

_mesh = plsc.VectorSubcoreMesh(
    core_axis_name="c", subcore_axis_name="s", num_cores=NC, num_subcores=NS
)


@functools.partial(
    pl.kernel,
    out_type=jax.ShapeDtypeStruct((NC * N,), jnp.float32),
    mesh=_mesh,
    scratch_types=(
        [pltpu.VMEM((CHUNK,), jnp.float32), pltpu.VMEM((RPT,), jnp.float32)]
        + [pltpu.VMEM((CHUNK,), jnp.int32)] * KBUF
        + [pltpu.SemaphoreType.DMA] * (2 * KBUF)
        + [pltpu.VMEM_SHARED((NACC,), jnp.float32)]
    ),
)
def _sc_deg(dst_hbm, out_hbm, ones_v, buf_v, *rest):
    dstb = rest[:KBUF]
    lsem = rest[KBUF:2 * KBUF]
    ssem = rest[2 * KBUF:3 * KBUF]
    deg_sh = rest[3 * KBUF]

    c = lax.axis_index("c")
    s = lax.axis_index("s")
    w = s * NC + c
    base0 = w * EDGES_PER_W

    for j in range(CHUNK // 16):
        ones_v[pl.ds(j * 16, 16)] = jnp.ones((16,), jnp.float32)

    def zbody(r, _):
        buf_v[pl.ds(r * 16, 16)] = jnp.zeros((16,), jnp.float32)
        return 0

    lax.fori_loop(0, RPT // 16, zbody, 0)
    pltpu.sync_copy(buf_v, deg_sh.at[pl.ds(s * RPT, RPT)])
    plsc.subcore_barrier()

    for b in range(KBUF):
        pltpu.async_copy(dst_hbm.at[pl.ds(base0 + b * CHUNK, CHUNK)],
                         dstb[b], lsem[b])

    def body(j, _):
        i0 = j * KBUF
        for b in range(KBUF):
            pltpu.make_async_copy(dst_hbm.at[pl.ds(base0, CHUNK)], dstb[b],
                                  lsem[b]).wait()
            pltpu.async_copy(ones_v, deg_sh.at[dstb[b]], ssem[b], add=True)
        for b in range(KBUF):
            i = i0 + b

            @pl.when(j < ROUNDS - 1)
            def _():
                pltpu.make_async_copy(ones_v, deg_sh.at[dstb[b]],
                                      ssem[b]).wait()
                pltpu.async_copy(
                    dst_hbm.at[pl.ds(base0 + (i + KBUF) * CHUNK, CHUNK)],
                    dstb[b], lsem[b])
        return 0

    lax.fori_loop(0, ROUNDS, body, 0)
    for b in range(KBUF):
        pltpu.make_async_copy(ones_v, deg_sh.at[dstb[b]], ssem[b]).wait()
    plsc.subcore_barrier()

    pltpu.sync_copy(deg_sh.at[pl.ds(s * RPT, RPT)], buf_v)

    @pl.when(s < NS - 1)
    def _():
        pltpu.sync_copy(buf_v, out_hbm.at[pl.ds(c * N + s * RPT, RPT)])

    @pl.when(s == NS - 1)
    def _():
        pltpu.sync_copy(buf_v.at[pl.ds(0, LASTC * CHUNK)],
                        out_hbm.at[pl.ds(c * N + (NS - 1) * RPT,
                                         LASTC * CHUNK)])


@functools.partial(
    pl.kernel,
    out_type=jax.ShapeDtypeStruct((NC * N, H), jnp.float32),
    mesh=_mesh,
    scratch_types=(
        [pltpu.VMEM((EDGES_PER_W,), jnp.int32)]
        + [pltpu.VMEM((CHUNK,), jnp.int32)] * KS
        + [pltpu.VMEM((CHUNK, H), jnp.float32)] * KS
        + [pltpu.SemaphoreType.DMA] * (3 * KS)
        + [pltpu.VMEM_SHARED((NACC, H), jnp.float32)]
    ),
)
def _sc_scatter(g_hbm, src_hbm, dst_hbm, out_hbm, src_all, *rest):
    dstb = rest[:KS]
    rows = rest[KS:2 * KS]
    gsem = rest[2 * KS:3 * KS]
    lsem = rest[3 * KS:4 * KS]
    ssem = rest[4 * KS:5 * KS]
    acc_sh = rest[5 * KS]

    c = lax.axis_index("c")
    s = lax.axis_index("s")
    w = s * NC + c
    base0 = w * EDGES_PER_W

    pltpu.sync_copy(src_hbm.at[pl.ds(base0, EDGES_PER_W)], src_all)

    def zbody(r, _):
        for j in range(H // 16):
            rows[0][r, pl.ds(j * 16, 16)] = jnp.zeros((16,), jnp.float32)
        return 0

    lax.fori_loop(0, CHUNK, zbody, 0)
    for k in range(RCOPY):
        pltpu.sync_copy(rows[0], acc_sh.at[pl.ds(s * RPT + k * CHUNK, CHUNK)])
    plsc.subcore_barrier()

    for b in range(KS):
        pltpu.async_copy(g_hbm.at[src_all.at[pl.ds(b * CHUNK, CHUNK)]],
                         rows[b], gsem[b])
        pltpu.async_copy(dst_hbm.at[pl.ds(base0 + b * CHUNK, CHUNK)],
                         dstb[b], lsem[b])

    def body(j, _):
        i0 = j * KS
        for b in range(KS):
            i = i0 + b
            pltpu.make_async_copy(g_hbm.at[src_all.at[pl.ds(0, CHUNK)]],
                                  rows[b], gsem[b]).wait()
            pltpu.make_async_copy(dst_hbm.at[pl.ds(base0, CHUNK)], dstb[b],
                                  lsem[b]).wait()
            pltpu.async_copy(rows[b], acc_sh.at[dstb[b]], ssem[b], add=True)
        for b in range(KS):
            i = i0 + b

            @pl.when(j < ROUNDS_S - 1)
            def _():
                pltpu.make_async_copy(rows[b], acc_sh.at[dstb[b]],
                                      ssem[b]).wait()
                pltpu.async_copy(
                    g_hbm.at[src_all.at[pl.ds((i + KS) * CHUNK, CHUNK)]],
                    rows[b], gsem[b])
                pltpu.async_copy(
                    dst_hbm.at[pl.ds(base0 + (i + KS) * CHUNK, CHUNK)],
                    dstb[b], lsem[b])
        return 0

    lax.fori_loop(0, ROUNDS_S, body, 0)

    for i in range(KS * ROUNDS_S, STEPS):
        b = (i - KS * ROUNDS_S) % KS
        pltpu.make_async_copy(rows[b], acc_sh.at[dstb[b]], ssem[b]).wait()
        pltpu.async_copy(g_hbm.at[src_all.at[pl.ds(i * CHUNK, CHUNK)]],
                         rows[b], gsem[b])
        pltpu.async_copy(dst_hbm.at[pl.ds(base0 + i * CHUNK, CHUNK)],
                         dstb[b], lsem[b])
        pltpu.make_async_copy(g_hbm.at[src_all.at[pl.ds(0, CHUNK)]],
                              rows[b], gsem[b]).wait()
        pltpu.make_async_copy(dst_hbm.at[pl.ds(base0, CHUNK)], dstb[b],
                              lsem[b]).wait()
        pltpu.async_copy(rows[b], acc_sh.at[dstb[b]], ssem[b], add=True)

    for b in range(KS):
        pltpu.make_async_copy(rows[b], acc_sh.at[dstb[b]], ssem[b]).wait()
    plsc.subcore_barrier()

    for k in range(RCOPY):
        if k < LASTC:
            pltpu.sync_copy(acc_sh.at[pl.ds(s * RPT + k * CHUNK, CHUNK)],
                            rows[0])
            pltpu.sync_copy(
                rows[0],
                out_hbm.at[pl.ds(c * N + s * RPT + k * CHUNK, CHUNK)],
            )
        else:
            @pl.when(s < NS - 1)
            def _():
                pltpu.sync_copy(acc_sh.at[pl.ds(s * RPT + k * CHUNK, CHUNK)],
                                rows[0])
                pltpu.sync_copy(
                    rows[0],
                    out_hbm.at[pl.ds(c * N + s * RPT + k * CHUNK, CHUNK)],
                )


_BN = 2000
_NB = N // _BN


_BE = 32000


def _tc_split_body(ei_ref, s_ref, d_ref):
    s_ref[...] = ei_ref[0]
    d_ref[...] = ei_ref[1]


def _tc_hw_body(x_ref, co_ref, wn_ref, bn_ref, wc_ref, bc_ref,
                wg1_ref, wg2_ref, hw_ref):
    hn = jnp.dot(x_ref[...], wn_ref[...], preferred_element_type=jnp.float32)
    hn = hn + bn_ref[...]
    hc = jnp.dot(co_ref[...], wc_ref[...], preferred_element_type=jnp.float32)
    hc = hc + bc_ref[...]
    hw_ref[...] = (jnp.dot(hn, wg1_ref[...], preferred_element_type=jnp.float32)
                   + jnp.dot(hc, wg2_ref[...], preferred_element_type=jnp.float32))


def _tc_scale_body(hw_ref, d0_ref, d1_ref, g_ref):
    dinv = lax.rsqrt(d0_ref[...] + d1_ref[...] + 1.0)
    g_ref[...] = dinv * hw_ref[...]


def _tc_final_body(a0_ref, a1_ref, g_ref, d0_ref, d1_ref, bg_ref, wf_ref,
                   bf_ref, out_ref):
    ssum = a0_ref[...] + a1_ref[...] + g_ref[...]
    dinv = lax.rsqrt(d0_ref[...] + d1_ref[...] + 1.0)
    h = jnp.maximum(dinv * ssum + bg_ref[...], 0.0)
    out_ref[...] = (jnp.dot(h, wf_ref[...], preferred_element_type=jnp.float32)
                    + bf_ref[...])


def _row_block(cols, off=0):
    return pl.BlockSpec((_BN, cols), lambda i, o=off: (i + o, 0))


def _whole(shape):
    return pl.BlockSpec(shape, lambda i: tuple(0 for _ in shape))


def kernel(x, edge_index, initial_coords, W_node, b_node, W_coord, b_coord,
           W_gcn, b_gcn, W_force, b_force):
    src, dst = pl.pallas_call(
        _tc_split_body,
        grid=(1,),
        in_specs=[
            pl.BlockSpec((2, E), lambda i: (0, 0)),
        ],
        out_specs=[
            pl.BlockSpec((E,), lambda i: (0,)),
            pl.BlockSpec((E,), lambda i: (0,)),
        ],
        out_shape=[
            jax.ShapeDtypeStruct((E,), jnp.int32),
            jax.ShapeDtypeStruct((E,), jnp.int32),
        ],
    )(edge_index)

    deg_p = _sc_deg(dst).reshape(NC * N, 1)

    hw = pl.pallas_call(
        _tc_hw_body,
        grid=(_NB,),
        in_specs=[
            _row_block(D_IN),
            _row_block(2),
            _whole((D_IN, H)),
            _whole((1, H)),
            _whole((2, HC)),
            _whole((1, HC)),
            _whole((H, H)),
            _whole((HC, H)),
        ],
        out_specs=_row_block(H),
        out_shape=jax.ShapeDtypeStruct((N, H), jnp.float32),
    )(x, initial_coords, W_node, b_node.reshape(1, H),
      W_coord, b_coord.reshape(1, HC), W_gcn[:H], W_gcn[H:])

    g = pl.pallas_call(
        _tc_scale_body,
        grid=(_NB,),
        in_specs=[
            _row_block(H),
            _row_block(1),
            _row_block(1, off=_NB),
        ],
        out_specs=_row_block(H),
        out_shape=jax.ShapeDtypeStruct((N, H), jnp.float32),
    )(hw, deg_p, deg_p)

    acc_p = _sc_scatter(g, src, dst)

    out = pl.pallas_call(
        _tc_final_body,
        grid=(_NB,),
        in_specs=[
            _row_block(H),
            _row_block(H, off=_NB),
            _row_block(H),
            _row_block(1),
            _row_block(1, off=_NB),
            _whole((1, H)),
            _whole((H, 2)),
            _whole((1, 2)),
        ],
        out_specs=_row_block(2),
        out_shape=jax.ShapeDtypeStruct((N, 2), jnp.float32),
    )(acc_p, acc_p, g, deg_p, deg_p, b_gcn.reshape(1, H), W_force,
      b_force.reshape(1, 2))

    return out


# speedup vs baseline: 40.9237x; 1.0152x over previous
import functools

import jax
import jax.numpy as jnp
from jax import lax
from jax.experimental import pallas as pl
from jax.experimental.pallas import tpu as pltpu
from jax.experimental.pallas import tpu_sc as plsc

N = 10000
E = 320000
D_IN = 128
H = 128
HC = 32

NC = 2
NS = 16
NW = NC * NS
EDGES_PER_W = E // NW
CHUNK = 80
STEPS = EDGES_PER_W // CHUNK
NACC = 10240
RPT = NACC // NS
RCOPY = RPT // CHUNK
LASTC = (N - (NS - 1) * RPT) // CHUNK

KBUF = 5
ROUNDS = STEPS // KBUF
KS = 3
ROUNDS_S = 40

_mesh = plsc.VectorSubcoreMesh(
    core_axis_name="c", subcore_axis_name="s", num_cores=NC, num_subcores=NS
)


@functools.partial(
    pl.kernel,
    out_type=jax.ShapeDtypeStruct((NC * N,), jnp.float32),
    mesh=_mesh,
    scratch_types=(
        [pltpu.VMEM((CHUNK,), jnp.float32), pltpu.VMEM((RPT,), jnp.float32)]
        + [pltpu.VMEM((CHUNK,), jnp.int32)] * KBUF
        + [pltpu.SemaphoreType.DMA] * (2 * KBUF)
        + [pltpu.VMEM_SHARED((NACC,), jnp.float32)]
    ),
)
def _sc_deg(dst_hbm, out_hbm, ones_v, buf_v, *rest):
    dstb = rest[:KBUF]
    lsem = rest[KBUF:2 * KBUF]
    ssem = rest[2 * KBUF:3 * KBUF]
    deg_sh = rest[3 * KBUF]

    c = lax.axis_index("c")
    s = lax.axis_index("s")
    w = s * NC + c
    base0 = w * EDGES_PER_W

    for j in range(CHUNK // 16):
        ones_v[pl.ds(j * 16, 16)] = jnp.ones((16,), jnp.float32)

    def zbody(r, _):
        buf_v[pl.ds(r * 16, 16)] = jnp.zeros((16,), jnp.float32)
        return 0

    lax.fori_loop(0, RPT // 16, zbody, 0)
    pltpu.sync_copy(buf_v, deg_sh.at[pl.ds(s * RPT, RPT)])
    plsc.subcore_barrier()

    for b in range(KBUF):
        pltpu.async_copy(dst_hbm.at[pl.ds(base0 + b * CHUNK, CHUNK)],
                         dstb[b], lsem[b])

    def body(j, _):
        i0 = j * KBUF
        for b in range(KBUF):
            pltpu.make_async_copy(dst_hbm.at[pl.ds(base0, CHUNK)], dstb[b],
                                  lsem[b]).wait()
            pltpu.async_copy(ones_v, deg_sh.at[dstb[b]], ssem[b], add=True)
        for b in range(KBUF):
            i = i0 + b

            @pl.when(j < ROUNDS - 1)
            def _():
                pltpu.make_async_copy(ones_v, deg_sh.at[dstb[b]],
                                      ssem[b]).wait()
                pltpu.async_copy(
                    dst_hbm.at[pl.ds(base0 + (i + KBUF) * CHUNK, CHUNK)],
                    dstb[b], lsem[b])
        return 0

    lax.fori_loop(0, ROUNDS, body, 0)
    for b in range(KBUF):
        pltpu.make_async_copy(ones_v, deg_sh.at[dstb[b]], ssem[b]).wait()
    plsc.subcore_barrier()

    pltpu.sync_copy(deg_sh.at[pl.ds(s * RPT, RPT)], buf_v)

    @pl.when(s < NS - 1)
    def _():
        pltpu.sync_copy(buf_v, out_hbm.at[pl.ds(c * N + s * RPT, RPT)])

    @pl.when(s == NS - 1)
    def _():
        pltpu.sync_copy(buf_v.at[pl.ds(0, LASTC * CHUNK)],
                        out_hbm.at[pl.ds(c * N + (NS - 1) * RPT,
                                         LASTC * CHUNK)])


@functools.partial(
    pl.kernel,
    out_type=jax.ShapeDtypeStruct((NC * N, H), jnp.float32),
    mesh=_mesh,
    scratch_types=(
        [pltpu.VMEM((EDGES_PER_W,), jnp.int32)]
        + [pltpu.VMEM((CHUNK,), jnp.int32)] * KS
        + [pltpu.VMEM((CHUNK, H), jnp.float32)] * KS
        + [pltpu.SemaphoreType.DMA] * (3 * KS)
        + [pltpu.VMEM_SHARED((NACC, H), jnp.float32)]
    ),
)
def _sc_scatter(g_hbm, src_hbm, dst_hbm, out_hbm, src_all, *rest):
    dstb = rest[:KS]
    rows = rest[KS:2 * KS]
    gsem = rest[2 * KS:3 * KS]
    lsem = rest[3 * KS:4 * KS]
    ssem = rest[4 * KS:5 * KS]
    acc_sh = rest[5 * KS]

    c = lax.axis_index("c")
    s = lax.axis_index("s")
    w = s * NC + c
    base0 = w * EDGES_PER_W

    pltpu.sync_copy(src_hbm.at[pl.ds(base0, EDGES_PER_W)], src_all)

    def zbody(r, _):
        for j in range(H // 16):
            rows[0][r, pl.ds(j * 16, 16)] = jnp.zeros((16,), jnp.float32)
        return 0

    lax.fori_loop(0, CHUNK, zbody, 0)
    for k in range(RCOPY):
        pltpu.async_copy(rows[0],
                         acc_sh.at[pl.ds(s * RPT + k * CHUNK, CHUNK)],
                         gsem[0])
    for k in range(RCOPY):
        pltpu.make_async_copy(rows[0], acc_sh.at[pl.ds(s * RPT, CHUNK)],
                              gsem[0]).wait()
    plsc.subcore_barrier()

    for b in range(KS):
        pltpu.async_copy(g_hbm.at[src_all.at[pl.ds(b * CHUNK, CHUNK)]],
                         rows[b], gsem[b])
        pltpu.async_copy(dst_hbm.at[pl.ds(base0 + b * CHUNK, CHUNK)],
                         dstb[b], lsem[b])

    def body(j, _):
        i0 = j * KS
        for b in range(KS):
            i = i0 + b
            pltpu.make_async_copy(g_hbm.at[src_all.at[pl.ds(0, CHUNK)]],
                                  rows[b], gsem[b]).wait()
            pltpu.make_async_copy(dst_hbm.at[pl.ds(base0, CHUNK)], dstb[b],
                                  lsem[b]).wait()
            pltpu.async_copy(rows[b], acc_sh.at[dstb[b]], ssem[b], add=True)
        for b in range(KS):
            i = i0 + b

            @pl.when(j < ROUNDS_S - 1)
            def _():
                pltpu.make_async_copy(rows[b], acc_sh.at[dstb[b]],
                                      ssem[b]).wait()
                pltpu.async_copy(
                    g_hbm.at[src_all.at[pl.ds((i + KS) * CHUNK, CHUNK)]],
                    rows[b], gsem[b])
                pltpu.async_copy(
                    dst_hbm.at[pl.ds(base0 + (i + KS) * CHUNK, CHUNK)],
                    dstb[b], lsem[b])
        return 0

    lax.fori_loop(0, ROUNDS_S, body, 0)

    for i in range(KS * ROUNDS_S, STEPS):
        b = (i - KS * ROUNDS_S) % KS
        pltpu.make_async_copy(rows[b], acc_sh.at[dstb[b]], ssem[b]).wait()
        pltpu.async_copy(g_hbm.at[src_all.at[pl.ds(i * CHUNK, CHUNK)]],
                         rows[b], gsem[b])
        pltpu.async_copy(dst_hbm.at[pl.ds(base0 + i * CHUNK, CHUNK)],
                         dstb[b], lsem[b])
        pltpu.make_async_copy(g_hbm.at[src_all.at[pl.ds(0, CHUNK)]],
                              rows[b], gsem[b]).wait()
        pltpu.make_async_copy(dst_hbm.at[pl.ds(base0, CHUNK)], dstb[b],
                              lsem[b]).wait()
        pltpu.async_copy(rows[b], acc_sh.at[dstb[b]], ssem[b], add=True)

    for b in range(KS):
        pltpu.make_async_copy(rows[b], acc_sh.at[dstb[b]], ssem[b]).wait()
    plsc.subcore_barrier()

    for k in range(RCOPY):
        b = k % KS

        def _wb(k=k, b=b):
            if k >= KS:
                pltpu.make_async_copy(
                    rows[b], out_hbm.at[pl.ds(c * N + s * RPT, CHUNK)],
                    lsem[b]).wait()
            pltpu.sync_copy(acc_sh.at[pl.ds(s * RPT + k * CHUNK, CHUNK)],
                            rows[b])
            pltpu.async_copy(
                rows[b],
                out_hbm.at[pl.ds(c * N + s * RPT + k * CHUNK, CHUNK)],
                lsem[b])

        if k < LASTC:
            _wb()
        else:
            pl.when(s < NS - 1)(_wb)

    for b in range(KS):
        pltpu.make_async_copy(rows[b],
                              out_hbm.at[pl.ds(c * N + s * RPT, CHUNK)],
                              lsem[b]).wait()


_BN = 2000
_NB = N // _BN


_BE = 32000


def _tc_split_body(ei_ref, s_ref, d_ref):
    s_ref[...] = ei_ref[0]
    d_ref[...] = ei_ref[1]


def _tc_hw_body(x_ref, co_ref, wn_ref, bn_ref, wc_ref, bc_ref,
                wg1_ref, wg2_ref, hw_ref):
    hn = jnp.dot(x_ref[...], wn_ref[...], preferred_element_type=jnp.float32)
    hn = hn + bn_ref[...]
    hc = jnp.dot(co_ref[...], wc_ref[...], preferred_element_type=jnp.float32)
    hc = hc + bc_ref[...]
    hw_ref[...] = (jnp.dot(hn, wg1_ref[...], preferred_element_type=jnp.float32)
                   + jnp.dot(hc, wg2_ref[...], preferred_element_type=jnp.float32))


def _tc_scale_body(hw_ref, d0_ref, d1_ref, g_ref):
    dinv = lax.rsqrt(d0_ref[...] + d1_ref[...] + 1.0)
    g_ref[...] = dinv * hw_ref[...]


def _tc_final_body(a0_ref, a1_ref, g_ref, d0_ref, d1_ref, bg_ref, wf_ref,
                   bf_ref, out_ref):
    ssum = a0_ref[...] + a1_ref[...] + g_ref[...]
    dinv = lax.rsqrt(d0_ref[...] + d1_ref[...] + 1.0)
    h = jnp.maximum(dinv * ssum + bg_ref[...], 0.0)
    out_ref[...] = (jnp.dot(h, wf_ref[...], preferred_element_type=jnp.float32)
                    + bf_ref[...])


def _row_block(cols, off=0):
    return pl.BlockSpec((_BN, cols), lambda i, o=off: (i + o, 0))


def _whole(shape):
    return pl.BlockSpec(shape, lambda i: tuple(0 for _ in shape))


def kernel(x, edge_index, initial_coords, W_node, b_node, W_coord, b_coord,
           W_gcn, b_gcn, W_force, b_force):
    src, dst = pl.pallas_call(
        _tc_split_body,
        grid=(1,),
        in_specs=[
            pl.BlockSpec((2, E), lambda i: (0, 0)),
        ],
        out_specs=[
            pl.BlockSpec((E,), lambda i: (0,)),
            pl.BlockSpec((E,), lambda i: (0,)),
        ],
        out_shape=[
            jax.ShapeDtypeStruct((E,), jnp.int32),
            jax.ShapeDtypeStruct((E,), jnp.int32),
        ],
    )(edge_index)

    deg_p = _sc_deg(dst).reshape(NC * N, 1)

    hw = pl.pallas_call(
        _tc_hw_body,
        grid=(_NB,),
        in_specs=[
            _row_block(D_IN),
            _row_block(2),
            _whole((D_IN, H)),
            _whole((1, H)),
            _whole((2, HC)),
            _whole((1, HC)),
            _whole((H, H)),
            _whole((HC, H)),
        ],
        out_specs=_row_block(H),
        out_shape=jax.ShapeDtypeStruct((N, H), jnp.float32),
    )(x, initial_coords, W_node, b_node.reshape(1, H),
      W_coord, b_coord.reshape(1, HC), W_gcn[:H], W_gcn[H:])

    g = pl.pallas_call(
        _tc_scale_body,
        grid=(_NB,),
        in_specs=[
            _row_block(H),
            _row_block(1),
            _row_block(1, off=_NB),
        ],
        out_specs=_row_block(H),
        out_shape=jax.ShapeDtypeStruct((N, H), jnp.float32),
    )(hw, deg_p, deg_p)

    acc_p = _sc_scatter(g, src, dst)

    out = pl.pallas_call(
        _tc_final_body,
        grid=(_NB,),
        in_specs=[
            _row_block(H),
            _row_block(H, off=_NB),
            _row_block(H),
            _row_block(1),
            _row_block(1, off=_NB),
            _whole((1, H)),
            _whole((H, 2)),
            _whole((1, 2)),
        ],
        out_specs=_row_block(2),
        out_shape=jax.ShapeDtypeStruct((N, 2), jnp.float32),
    )(acc_p, acc_p, g, deg_p, deg_p, b_gcn.reshape(1, H), W_force,
      b_force.reshape(1, 2))

    return out
